# baseline (speedup 1.0000x reference)
import functools

import jax
import jax.numpy as jnp
from jax import lax
from jax.experimental import pallas as pl
from jax.experimental.pallas import tpu as pltpu
from jax.experimental.pallas import tpu_sc as plsc

NCORES = 2
NSUB = 16
NW = NCORES * NSUB
CHUNK = 128
KCH = 16
NBUF = 3
DEGW = 16


_SC_MESH = plsc.VectorSubcoreMesh(core_axis_name="c", subcore_axis_name="s")
_SC_PARAMS = pltpu.CompilerParams(use_tc_tiling_on_sc=False)


def _make_sc_segsum(n, n_pad, d, n_chunks, col_split, with_deg):
    rows_per_sub = n_pad // NSUB
    trows_per_sub = n // NSUB

    out_type = [jax.ShapeDtypeStruct((NCORES, n_pad, d), jnp.float32)]
    scratch = [
        pltpu.VMEM((2 * KCH, CHUNK), jnp.int32),
        [pltpu.VMEM((CHUNK, d), jnp.float32)
         for _ in range(NBUF)],
        pltpu.VMEM_SHARED((n, d), jnp.float32),
        pltpu.VMEM_SHARED((n_pad, d), jnp.float32),
        pltpu.SemaphoreType.DMA,
        pltpu.SemaphoreType.DMA,
    ]
    if with_deg:
        out_type.append(
            jax.ShapeDtypeStruct((NCORES, n_pad, DEGW), jnp.float32))
        scratch += [
            pltpu.VMEM((CHUNK, DEGW), jnp.float32),
            pltpu.VMEM_SHARED((n_pad, DEGW), jnp.float32),
            pltpu.SemaphoreType.DMA,
        ]

    def body(table_hbm, idx_hbm, zeros_hbm, zeros16_hbm, ones_hbm,
             agg_out, *rest):
        if with_deg:
            (deg_out, idx_v, rows, table_sh, agg_sh, gsem, ssem,
             ones_v, deg_sh, dsem) = rest
        else:
            idx_v, rows, table_sh, agg_sh, gsem, ssem = rest
        c = lax.axis_index("c")
        s = lax.axis_index("s")
        sl = pl.ds(s * rows_per_sub, rows_per_sub)
        tsl = pl.ds(s * trows_per_sub, trows_per_sub)
        if col_split:
            pltpu.sync_copy(table_hbm.at[tsl, pl.ds(c * d, d)],
                            table_sh.at[tsl])
        else:
            pltpu.sync_copy(table_hbm.at[tsl], table_sh.at[tsl])
        pltpu.sync_copy(zeros_hbm.at[sl], agg_sh.at[sl])
        if with_deg:
            pltpu.sync_copy(zeros16_hbm.at[sl], deg_sh.at[sl])
            pltpu.sync_copy(ones_hbm, ones_v)
        plsc.subcore_barrier()

        @pl.loop(0, n_chunks // KCH)
        def _(g):
            if col_split:
                pltpu.sync_copy(idx_hbm.at[s, g], idx_v)
            else:
                pltpu.sync_copy(idx_hbm.at[c, s, g], idx_v)
            gathers = [
                pltpu.async_copy(table_sh.at[idx_v.at[0]], rows[0], gsem)]
            scatters = []
            dscats = []
            for j in range(KCH):
                gathers[j].wait()
                scatters.append(pltpu.async_copy(
                    rows[j % NBUF], agg_sh.at[idx_v.at[2 * j + 1]], ssem,
                    add=True))
                if with_deg:
                    dscats.append(pltpu.async_copy(
                        ones_v, deg_sh.at[idx_v.at[2 * j + 1]], dsem,
                        add=True))
                if j + 1 < KCH:
                    if j >= NBUF - 1:
                        scatters[j - (NBUF - 1)].wait()
                    gathers.append(pltpu.async_copy(
                        table_sh.at[idx_v.at[2 * (j + 1)]],
                        rows[(j + 1) % NBUF], gsem))
            for j in range(KCH - NBUF, KCH):
                scatters[j].wait()
            for dsc in dscats:
                dsc.wait()

        plsc.subcore_barrier()
        pltpu.sync_copy(agg_sh.at[sl], agg_out.at[c, sl])
        if with_deg:
            pltpu.sync_copy(deg_sh.at[sl], deg_out.at[c, sl])

    return pl.kernel(body, out_type=tuple(out_type) if with_deg
                     else out_type[0],
                     mesh=_SC_MESH, scratch_types=scratch,
                     compiler_params=_SC_PARAMS)


def _mm2_kernel(x_ref, wa_ref, wb_ref, oa_ref, ob_ref):
    x = x_ref[...]
    oa_ref[...] = jnp.dot(x, wa_ref[...], preferred_element_type=jnp.float32)
    ob_ref[...] = jnp.dot(x, wb_ref[...], preferred_element_type=jnp.float32)


def _layer1_kernel(xs_ref, a0_ref, a1_ref, d0_ref, b_ref,
                   ws2_ref, wn2_ref, hs_ref, hn_ref):
    rdeg = 1.0 / jnp.maximum(d0_ref[:, 0], 1.0)
    agg = jnp.concatenate([a0_ref[...], a1_ref[...]], axis=1) * rdeg[:, None]
    h = jnp.maximum(xs_ref[...] + agg + b_ref[...], 0.0)
    hs_ref[...] = jnp.dot(h, ws2_ref[...], preferred_element_type=jnp.float32)
    hn_ref[...] = jnp.dot(h, wn2_ref[...], preferred_element_type=jnp.float32)


def _layer2_kernel(hs_ref, a0_ref, a1_ref, d0_ref, b_ref, o_ref):
    rdeg = 1.0 / jnp.maximum(d0_ref[:, 0], 1.0)
    agg = (a0_ref[...] + a1_ref[...]) * rdeg[:, None]
    logits = hs_ref[...] + agg + b_ref[...]
    m = jnp.max(logits, axis=-1, keepdims=True)
    z = logits - m
    lse = jnp.log(jnp.sum(jnp.exp(z), axis=-1, keepdims=True))
    o_ref[...] = z - lse


def kernel(x, edge_index, W_self1, W_neigh1, b1, W_self2, W_neigh2, b2):
    n, d_in = x.shape
    d_h = W_self1.shape[1]
    d_out = W_self2.shape[1]
    e = edge_index.shape[1]

    n_pad = ((n + NSUB - 1) // NSUB + 7) // 8 * 8 * NSUB
    group = KCH * CHUNK
    src_flat = edge_index[0].astype(jnp.int32)
    dst_flat = edge_index[1].astype(jnp.int32)

    def _partition(nw):
        e_per = e // nw
        n_chunks = (e_per + group - 1) // group * KCH
        pad = n_chunks * CHUNK - e_per
        s = jnp.pad(src_flat.reshape(nw, e_per), ((0, 0), (0, pad)))
        d = jnp.pad(dst_flat.reshape(nw, e_per), ((0, 0), (0, pad)),
                    constant_values=n)
        return (s.reshape(nw, n_chunks, CHUNK),
                d.reshape(nw, n_chunks, CHUNK), n_chunks)

    src1, dst1, nc1 = _partition(NSUB)
    src2, dst2, nc2 = _partition(NW)
    idx1 = jnp.stack([src1, dst1], axis=2).reshape(
        NSUB, nc1 // KCH, 2 * KCH, CHUNK)
    idx2 = jnp.stack([src2, dst2], axis=2).reshape(
        NCORES, NSUB, nc2 // KCH, 2 * KCH, CHUNK)

    d_half = d_h // NCORES
    zeros_hh = jnp.zeros((n_pad, d_half), jnp.float32)
    zeros_o = jnp.zeros((n_pad, d_out), jnp.float32)
    zeros16 = jnp.zeros((n_pad, DEGW), jnp.float32)
    ones16 = jnp.ones((CHUNK, DEGW), jnp.float32)

    grid_r = 1000
    grid = (n // grid_r,)

    xs1, xn1 = pl.pallas_call(
        _mm2_kernel,
        grid=grid,
        in_specs=[
            pl.BlockSpec((grid_r, d_in), lambda i: (i, 0)),
            pl.BlockSpec((d_in, d_h), lambda i: (0, 0)),
            pl.BlockSpec((d_in, d_h), lambda i: (0, 0)),
        ],
        out_specs=[
            pl.BlockSpec((grid_r, d_h), lambda i: (i, 0)),
            pl.BlockSpec((grid_r, d_h), lambda i: (i, 0)),
        ],
        out_shape=[
            jax.ShapeDtypeStruct((n, d_h), jnp.float32),
            jax.ShapeDtypeStruct((n, d_h), jnp.float32),
        ],
    )(x, W_self1, W_neigh1)

    agg1p, degp = _make_sc_segsum(n, n_pad, d_half, nc1, True, True)(
        xn1, idx1, zeros_hh, zeros16, ones16)

    hs2, hn2 = pl.pallas_call(
        _layer1_kernel,
        grid=grid,
        in_specs=[
            pl.BlockSpec((grid_r, d_h), lambda i: (i, 0)),
            pl.BlockSpec((None, grid_r, d_half), lambda i: (0, i, 0)),
            pl.BlockSpec((None, grid_r, d_half), lambda i: (1, i, 0)),
            pl.BlockSpec((None, grid_r, DEGW), lambda i: (0, i, 0)),
            pl.BlockSpec((1, d_h), lambda i: (0, 0)),
            pl.BlockSpec((d_h, d_out), lambda i: (0, 0)),
            pl.BlockSpec((d_h, d_out), lambda i: (0, 0)),
        ],
        out_specs=[
            pl.BlockSpec((grid_r, d_out), lambda i: (i, 0)),
            pl.BlockSpec((grid_r, d_out), lambda i: (i, 0)),
        ],
        out_shape=[
            jax.ShapeDtypeStruct((n, d_out), jnp.float32),
            jax.ShapeDtypeStruct((n, d_out), jnp.float32),
        ],
    )(xs1, agg1p, agg1p, degp, b1.reshape(1, d_h), W_self2, W_neigh2)

    agg2p = _make_sc_segsum(n, n_pad, d_out, nc2, False, False)(
        hn2, idx2, zeros_o, zeros16, ones16)

    out = pl.pallas_call(
        _layer2_kernel,
        grid=grid,
        in_specs=[
            pl.BlockSpec((grid_r, d_out), lambda i: (i, 0)),
            pl.BlockSpec((None, grid_r, d_out), lambda i: (0, i, 0)),
            pl.BlockSpec((None, grid_r, d_out), lambda i: (1, i, 0)),
            pl.BlockSpec((None, grid_r, DEGW), lambda i: (0, i, 0)),
            pl.BlockSpec((1, d_out), lambda i: (0, 0)),
        ],
        out_specs=pl.BlockSpec((grid_r, d_out), lambda i: (i, 0)),
        out_shape=jax.ShapeDtypeStruct((n, d_out), jnp.float32),
    )(hs2, agg2p, agg2p, degp, b2.reshape(1, d_out))

    return out


# speedup vs baseline: 1.2525x; 1.2525x over previous
import functools

import jax
import jax.numpy as jnp
from jax import lax
from jax.experimental import pallas as pl
from jax.experimental.pallas import tpu as pltpu
from jax.experimental.pallas import tpu_sc as plsc

NCORES = 2
NSUB = 16
NW = NCORES * NSUB
CHUNK = 128
KCH = 16
NBUF = 3
DEGW = 16


_SC_MESH = plsc.VectorSubcoreMesh(core_axis_name="c", subcore_axis_name="s")
_SC_PARAMS = pltpu.CompilerParams(use_tc_tiling_on_sc=False)


def _make_sc_segsum(n, n_pad, d, n_chunks, col_split, with_deg):
    rows_per_sub = n_pad // NSUB
    trows_per_sub = n // NSUB

    out_type = [jax.ShapeDtypeStruct((NCORES, n_pad, d), jnp.float32)]
    scratch = [
        pltpu.VMEM((2 * KCH, CHUNK), jnp.int32),
        [pltpu.VMEM((CHUNK, d), jnp.float32)
         for _ in range(NBUF)],
        pltpu.VMEM_SHARED((n, d), jnp.float32),
        pltpu.VMEM_SHARED((n_pad, d), jnp.float32),
        pltpu.SemaphoreType.DMA,
        pltpu.SemaphoreType.DMA,
    ]
    if with_deg:
        out_type.append(
            jax.ShapeDtypeStruct((NCORES, n_pad, DEGW), jnp.float32))
        scratch += [
            pltpu.VMEM((CHUNK, DEGW), jnp.float32),
            pltpu.VMEM_SHARED((n_pad, DEGW), jnp.float32),
            pltpu.SemaphoreType.DMA,
        ]

    def body(table_hbm, idx_hbm, zeros_hbm, zeros16_hbm, ones_hbm,
             agg_out, *rest):
        if with_deg:
            (deg_out, idx_v, rows, table_sh, agg_sh, gsem, ssem,
             ones_v, deg_sh, dsem) = rest
        else:
            idx_v, rows, table_sh, agg_sh, gsem, ssem = rest
        c = lax.axis_index("c")
        s = lax.axis_index("s")
        sl = pl.ds(s * rows_per_sub, rows_per_sub)
        tsl = pl.ds(s * trows_per_sub, trows_per_sub)
        if col_split:
            pltpu.sync_copy(table_hbm.at[tsl, pl.ds(c * d, d)],
                            table_sh.at[tsl])
        else:
            pltpu.sync_copy(table_hbm.at[tsl], table_sh.at[tsl])
        pltpu.sync_copy(zeros_hbm.at[sl], agg_sh.at[sl])
        if with_deg:
            pltpu.sync_copy(zeros16_hbm.at[sl], deg_sh.at[sl])
            pltpu.sync_copy(ones_hbm, ones_v)
        plsc.subcore_barrier()

        @pl.loop(0, n_chunks // KCH)
        def _(g):
            if col_split:
                pltpu.sync_copy(idx_hbm.at[s, g], idx_v)
            else:
                pltpu.sync_copy(idx_hbm.at[c, s, g], idx_v)
            gathers = [
                pltpu.async_copy(table_sh.at[idx_v.at[0]], rows[0], gsem)]
            scatters = []
            dscats = []
            for j in range(KCH):
                gathers[j].wait()
                scatters.append(pltpu.async_copy(
                    rows[j % NBUF], agg_sh.at[idx_v.at[2 * j + 1]], ssem,
                    add=True))
                if with_deg:
                    dscats.append(pltpu.async_copy(
                        ones_v, deg_sh.at[idx_v.at[2 * j + 1]], dsem,
                        add=True))
                if j + 1 < KCH:
                    if j >= NBUF - 1:
                        scatters[j - (NBUF - 1)].wait()
                    gathers.append(pltpu.async_copy(
                        table_sh.at[idx_v.at[2 * (j + 1)]],
                        rows[(j + 1) % NBUF], gsem))
            for j in range(KCH - NBUF, KCH):
                scatters[j].wait()
            for dsc in dscats:
                dsc.wait()

        plsc.subcore_barrier()
        pltpu.sync_copy(agg_sh.at[sl], agg_out.at[c, sl])
        if with_deg:
            pltpu.sync_copy(deg_sh.at[sl], deg_out.at[c, sl])

    return pl.kernel(body, out_type=tuple(out_type) if with_deg
                     else out_type[0],
                     mesh=_SC_MESH, scratch_types=scratch,
                     compiler_params=_SC_PARAMS)


def _make_sc_deg(n_pad, n_chunks):
    rows_per_sub = n_pad // NSUB

    out_type = jax.ShapeDtypeStruct((NCORES, n_pad, DEGW), jnp.float32)
    scratch = [
        pltpu.VMEM((2 * KCH, CHUNK), jnp.int32),
        pltpu.VMEM((CHUNK, DEGW), jnp.float32),
        pltpu.VMEM_SHARED((n_pad, DEGW), jnp.float32),
        pltpu.SemaphoreType.DMA,
    ]

    def body(idx_hbm, zeros16_hbm, ones_hbm, deg_out,
             idx_v, ones_v, deg_sh, ssem):
        c = lax.axis_index("c")
        s = lax.axis_index("s")
        sl = pl.ds(s * rows_per_sub, rows_per_sub)
        pltpu.sync_copy(zeros16_hbm.at[sl], deg_sh.at[sl])
        pltpu.sync_copy(ones_hbm, ones_v)
        plsc.subcore_barrier()

        @pl.loop(0, n_chunks // KCH)
        def _(g):
            pltpu.sync_copy(idx_hbm.at[c, s, g], idx_v)
            scatters = [
                pltpu.async_copy(ones_v, deg_sh.at[idx_v.at[2 * j + 1]],
                                 ssem, add=True)
                for j in range(KCH)]
            for sc in scatters:
                sc.wait()

        plsc.subcore_barrier()
        pltpu.sync_copy(deg_sh.at[sl], deg_out.at[c, sl])

    return pl.kernel(body, out_type=out_type, mesh=_SC_MESH,
                     scratch_types=scratch, compiler_params=_SC_PARAMS)


def _mm2_kernel(x_ref, wa_ref, wb_ref, oa_ref, ob_ref):
    x = x_ref[...]
    oa_ref[...] = jnp.dot(x, wa_ref[...], preferred_element_type=jnp.float32)
    ob_ref[...] = jnp.dot(x, wb_ref[...], preferred_element_type=jnp.float32)


def _layer1_kernel(xs_ref, a0_ref, a1_ref, d0_ref, d1_ref, b_ref,
                   ws2_ref, wn2_ref, hs_ref, hn_ref):
    rdeg = 1.0 / jnp.maximum(d0_ref[:, 0] + d1_ref[:, 0], 1.0)
    agg = jnp.concatenate([a0_ref[...], a1_ref[...]], axis=1) * rdeg[:, None]
    h = jnp.maximum(xs_ref[...] + agg + b_ref[...], 0.0)
    hs_ref[...] = jnp.dot(h, ws2_ref[...], preferred_element_type=jnp.float32)
    hn_ref[...] = jnp.dot(h, wn2_ref[...], preferred_element_type=jnp.float32)


def _layer2_kernel(hs_ref, a0_ref, a1_ref, d0_ref, d1_ref, b_ref, o_ref):
    rdeg = 1.0 / jnp.maximum(d0_ref[:, 0] + d1_ref[:, 0], 1.0)
    agg = (a0_ref[...] + a1_ref[...]) * rdeg[:, None]
    logits = hs_ref[...] + agg + b_ref[...]
    m = jnp.max(logits, axis=-1, keepdims=True)
    z = logits - m
    lse = jnp.log(jnp.sum(jnp.exp(z), axis=-1, keepdims=True))
    o_ref[...] = z - lse


def kernel(x, edge_index, W_self1, W_neigh1, b1, W_self2, W_neigh2, b2):
    n, d_in = x.shape
    d_h = W_self1.shape[1]
    d_out = W_self2.shape[1]
    e = edge_index.shape[1]

    n_pad = ((n + NSUB - 1) // NSUB + 7) // 8 * 8 * NSUB
    group = KCH * CHUNK
    src_flat = edge_index[0].astype(jnp.int32)
    dst_flat = edge_index[1].astype(jnp.int32)

    def _partition(nw):
        e_per = e // nw
        n_chunks = (e_per + group - 1) // group * KCH
        pad = n_chunks * CHUNK - e_per
        s = jnp.pad(src_flat.reshape(nw, e_per), ((0, 0), (0, pad)))
        d = jnp.pad(dst_flat.reshape(nw, e_per), ((0, 0), (0, pad)),
                    constant_values=n)
        return (s.reshape(nw, n_chunks, CHUNK),
                d.reshape(nw, n_chunks, CHUNK), n_chunks)

    src1, dst1, nc1 = _partition(NSUB)
    src2, dst2, nc2 = _partition(NW)
    idx1 = jnp.stack([src1, dst1], axis=2).reshape(
        NSUB, nc1 // KCH, 2 * KCH, CHUNK)
    idx2 = jnp.stack([src2, dst2], axis=2).reshape(
        NCORES, NSUB, nc2 // KCH, 2 * KCH, CHUNK)

    d_half = d_h // NCORES
    zeros_hh = jnp.zeros((n_pad, d_half), jnp.float32)
    zeros_o = jnp.zeros((n_pad, d_out), jnp.float32)
    zeros16 = jnp.zeros((n_pad, DEGW), jnp.float32)
    ones16 = jnp.ones((CHUNK, DEGW), jnp.float32)

    grid_r = 1000
    grid = (n // grid_r,)

    xs1, xn1 = pl.pallas_call(
        _mm2_kernel,
        grid=grid,
        in_specs=[
            pl.BlockSpec((grid_r, d_in), lambda i: (i, 0)),
            pl.BlockSpec((d_in, d_h), lambda i: (0, 0)),
            pl.BlockSpec((d_in, d_h), lambda i: (0, 0)),
        ],
        out_specs=[
            pl.BlockSpec((grid_r, d_h), lambda i: (i, 0)),
            pl.BlockSpec((grid_r, d_h), lambda i: (i, 0)),
        ],
        out_shape=[
            jax.ShapeDtypeStruct((n, d_h), jnp.float32),
            jax.ShapeDtypeStruct((n, d_h), jnp.float32),
        ],
    )(x, W_self1, W_neigh1)

    degp2 = _make_sc_deg(n_pad, nc2)(idx2, zeros16, ones16)

    agg1p = _make_sc_segsum(n, n_pad, d_half, nc1, True, False)(
        xn1, idx1, zeros_hh, zeros16, ones16)

    hs2, hn2 = pl.pallas_call(
        _layer1_kernel,
        grid=grid,
        in_specs=[
            pl.BlockSpec((grid_r, d_h), lambda i: (i, 0)),
            pl.BlockSpec((None, grid_r, d_half), lambda i: (0, i, 0)),
            pl.BlockSpec((None, grid_r, d_half), lambda i: (1, i, 0)),
            pl.BlockSpec((None, grid_r, DEGW), lambda i: (0, i, 0)),
            pl.BlockSpec((None, grid_r, DEGW), lambda i: (1, i, 0)),
            pl.BlockSpec((1, d_h), lambda i: (0, 0)),
            pl.BlockSpec((d_h, d_out), lambda i: (0, 0)),
            pl.BlockSpec((d_h, d_out), lambda i: (0, 0)),
        ],
        out_specs=[
            pl.BlockSpec((grid_r, d_out), lambda i: (i, 0)),
            pl.BlockSpec((grid_r, d_out), lambda i: (i, 0)),
        ],
        out_shape=[
            jax.ShapeDtypeStruct((n, d_out), jnp.float32),
            jax.ShapeDtypeStruct((n, d_out), jnp.float32),
        ],
    )(xs1, agg1p, agg1p, degp2, degp2, b1.reshape(1, d_h), W_self2, W_neigh2)

    agg2p = _make_sc_segsum(n, n_pad, d_out, nc2, False, False)(
        hn2, idx2, zeros_o, zeros16, ones16)

    out = pl.pallas_call(
        _layer2_kernel,
        grid=grid,
        in_specs=[
            pl.BlockSpec((grid_r, d_out), lambda i: (i, 0)),
            pl.BlockSpec((None, grid_r, d_out), lambda i: (0, i, 0)),
            pl.BlockSpec((None, grid_r, d_out), lambda i: (1, i, 0)),
            pl.BlockSpec((None, grid_r, DEGW), lambda i: (0, i, 0)),
            pl.BlockSpec((None, grid_r, DEGW), lambda i: (1, i, 0)),
            pl.BlockSpec((1, d_out), lambda i: (0, 0)),
        ],
        out_specs=pl.BlockSpec((grid_r, d_out), lambda i: (i, 0)),
        out_shape=jax.ShapeDtypeStruct((n, d_out), jnp.float32),
    )(hs2, agg2p, agg2p, degp2, degp2, b2.reshape(1, d_out))

    return out


# speedup vs baseline: 1.3043x; 1.0414x over previous
import functools

import jax
import jax.numpy as jnp
from jax import lax
from jax.experimental import pallas as pl
from jax.experimental.pallas import tpu as pltpu
from jax.experimental.pallas import tpu_sc as plsc

NCORES = 2
NSUB = 16
NW = NCORES * NSUB
CHUNK = 128
KCH = 40
NBUF = 3
DEGW = 16


_SC_MESH = plsc.VectorSubcoreMesh(core_axis_name="c", subcore_axis_name="s")
_SC_PARAMS = pltpu.CompilerParams(use_tc_tiling_on_sc=False)


def _make_sc_segsum(n, n_pad, d, n_chunks, col_split, with_deg):
    rows_per_sub = n_pad // NSUB
    trows_per_sub = n // NSUB

    out_type = [jax.ShapeDtypeStruct((NCORES, n_pad, d), jnp.float32)]
    scratch = [
        pltpu.VMEM((2 * KCH, CHUNK), jnp.int32),
        [pltpu.VMEM((CHUNK, d), jnp.float32)
         for _ in range(NBUF)],
        pltpu.VMEM_SHARED((n, d), jnp.float32),
        pltpu.VMEM_SHARED((n_pad, d), jnp.float32),
        pltpu.SemaphoreType.DMA,
        pltpu.SemaphoreType.DMA,
    ]
    if with_deg:
        out_type.append(
            jax.ShapeDtypeStruct((NCORES, n_pad, DEGW), jnp.float32))
        scratch += [
            pltpu.VMEM((CHUNK, DEGW), jnp.float32),
            pltpu.VMEM_SHARED((n_pad, DEGW), jnp.float32),
            pltpu.SemaphoreType.DMA,
        ]

    def body(table_hbm, idx_hbm, zeros_hbm, zeros16_hbm, ones_hbm,
             agg_out, *rest):
        if with_deg:
            (deg_out, idx_v, rows, table_sh, agg_sh, gsem, ssem,
             ones_v, deg_sh, dsem) = rest
        else:
            idx_v, rows, table_sh, agg_sh, gsem, ssem = rest
        c = lax.axis_index("c")
        s = lax.axis_index("s")
        sl = pl.ds(s * rows_per_sub, rows_per_sub)
        tsl = pl.ds(s * trows_per_sub, trows_per_sub)
        if col_split:
            pltpu.sync_copy(table_hbm.at[tsl, pl.ds(c * d, d)],
                            table_sh.at[tsl])
        else:
            pltpu.sync_copy(table_hbm.at[tsl], table_sh.at[tsl])
        pltpu.sync_copy(zeros_hbm.at[sl], agg_sh.at[sl])
        if with_deg:
            pltpu.sync_copy(zeros16_hbm.at[sl], deg_sh.at[sl])
            pltpu.sync_copy(ones_hbm, ones_v)
        plsc.subcore_barrier()

        @pl.loop(0, n_chunks // KCH)
        def _(g):
            if col_split:
                pltpu.sync_copy(idx_hbm.at[s, g], idx_v)
            else:
                pltpu.sync_copy(idx_hbm.at[c, s, g], idx_v)
            gathers = [
                pltpu.async_copy(table_sh.at[idx_v.at[0]], rows[0], gsem)]
            scatters = []
            dscats = []
            for j in range(KCH):
                gathers[j].wait()
                scatters.append(pltpu.async_copy(
                    rows[j % NBUF], agg_sh.at[idx_v.at[2 * j + 1]], ssem,
                    add=True))
                if with_deg:
                    dscats.append(pltpu.async_copy(
                        ones_v, deg_sh.at[idx_v.at[2 * j + 1]], dsem,
                        add=True))
                if j + 1 < KCH:
                    if j >= NBUF - 1:
                        scatters[j - (NBUF - 1)].wait()
                    gathers.append(pltpu.async_copy(
                        table_sh.at[idx_v.at[2 * (j + 1)]],
                        rows[(j + 1) % NBUF], gsem))
            for j in range(KCH - NBUF, KCH):
                scatters[j].wait()
            for dsc in dscats:
                dsc.wait()

        plsc.subcore_barrier()
        pltpu.sync_copy(agg_sh.at[sl], agg_out.at[c, sl])
        if with_deg:
            pltpu.sync_copy(deg_sh.at[sl], deg_out.at[c, sl])

    return pl.kernel(body, out_type=tuple(out_type) if with_deg
                     else out_type[0],
                     mesh=_SC_MESH, scratch_types=scratch,
                     compiler_params=_SC_PARAMS)


def _make_sc_deg(n_pad, n_chunks):
    rows_per_sub = n_pad // NSUB

    out_type = jax.ShapeDtypeStruct((NCORES, n_pad, DEGW), jnp.float32)
    scratch = [
        pltpu.VMEM((2 * KCH, CHUNK), jnp.int32),
        pltpu.VMEM((CHUNK, DEGW), jnp.float32),
        pltpu.VMEM_SHARED((n_pad, DEGW), jnp.float32),
        pltpu.SemaphoreType.DMA,
    ]

    def body(idx_hbm, zeros16_hbm, ones_hbm, deg_out,
             idx_v, ones_v, deg_sh, ssem):
        c = lax.axis_index("c")
        s = lax.axis_index("s")
        sl = pl.ds(s * rows_per_sub, rows_per_sub)
        pltpu.sync_copy(zeros16_hbm.at[sl], deg_sh.at[sl])
        pltpu.sync_copy(ones_hbm, ones_v)
        plsc.subcore_barrier()

        @pl.loop(0, n_chunks // KCH)
        def _(g):
            pltpu.sync_copy(idx_hbm.at[c, s, g], idx_v)
            scatters = [
                pltpu.async_copy(ones_v, deg_sh.at[idx_v.at[2 * j + 1]],
                                 ssem, add=True)
                for j in range(KCH)]
            for sc in scatters:
                sc.wait()

        plsc.subcore_barrier()
        pltpu.sync_copy(deg_sh.at[sl], deg_out.at[c, sl])

    return pl.kernel(body, out_type=out_type, mesh=_SC_MESH,
                     scratch_types=scratch, compiler_params=_SC_PARAMS)


def _mm2_kernel(x_ref, wa_ref, wb_ref, oa_ref, ob_ref):
    x = x_ref[...]
    oa_ref[...] = jnp.dot(x, wa_ref[...], preferred_element_type=jnp.float32)
    ob_ref[...] = jnp.dot(x, wb_ref[...], preferred_element_type=jnp.float32)


def _layer1_kernel(xs_ref, a0_ref, a1_ref, d0_ref, d1_ref, b_ref,
                   ws2_ref, wn2_ref, hs_ref, hn_ref):
    rdeg = 1.0 / jnp.maximum(d0_ref[:, 0] + d1_ref[:, 0], 1.0)
    agg = jnp.concatenate([a0_ref[...], a1_ref[...]], axis=1) * rdeg[:, None]
    h = jnp.maximum(xs_ref[...] + agg + b_ref[...], 0.0)
    hs_ref[...] = jnp.dot(h, ws2_ref[...], preferred_element_type=jnp.float32)
    hn_ref[...] = jnp.dot(h, wn2_ref[...], preferred_element_type=jnp.float32)


def _layer2_kernel(hs_ref, a0_ref, a1_ref, d0_ref, d1_ref, b_ref, o_ref):
    rdeg = 1.0 / jnp.maximum(d0_ref[:, 0] + d1_ref[:, 0], 1.0)
    agg = (a0_ref[...] + a1_ref[...]) * rdeg[:, None]
    logits = hs_ref[...] + agg + b_ref[...]
    m = jnp.max(logits, axis=-1, keepdims=True)
    z = logits - m
    lse = jnp.log(jnp.sum(jnp.exp(z), axis=-1, keepdims=True))
    o_ref[...] = z - lse


def kernel(x, edge_index, W_self1, W_neigh1, b1, W_self2, W_neigh2, b2):
    n, d_in = x.shape
    d_h = W_self1.shape[1]
    d_out = W_self2.shape[1]
    e = edge_index.shape[1]

    n_pad = ((n + NSUB - 1) // NSUB + 7) // 8 * 8 * NSUB
    group = KCH * CHUNK
    src_flat = edge_index[0].astype(jnp.int32)
    dst_flat = edge_index[1].astype(jnp.int32)

    def _partition(nw):
        e_per = e // nw
        n_chunks = (e_per + group - 1) // group * KCH
        pad = n_chunks * CHUNK - e_per
        s = jnp.pad(src_flat.reshape(nw, e_per), ((0, 0), (0, pad)))
        d = jnp.pad(dst_flat.reshape(nw, e_per), ((0, 0), (0, pad)),
                    constant_values=n)
        return (s.reshape(nw, n_chunks, CHUNK),
                d.reshape(nw, n_chunks, CHUNK), n_chunks)

    src1, dst1, nc1 = _partition(NSUB)
    src2, dst2, nc2 = _partition(NW)
    idx1 = jnp.stack([src1, dst1], axis=2).reshape(
        NSUB, nc1 // KCH, 2 * KCH, CHUNK)
    idx2 = jnp.stack([src2, dst2], axis=2).reshape(
        NCORES, NSUB, nc2 // KCH, 2 * KCH, CHUNK)

    d_half = d_h // NCORES
    zeros_hh = jnp.zeros((n_pad, d_half), jnp.float32)
    zeros_o = jnp.zeros((n_pad, d_out), jnp.float32)
    zeros16 = jnp.zeros((n_pad, DEGW), jnp.float32)
    ones16 = jnp.ones((CHUNK, DEGW), jnp.float32)

    grid_r = 1000
    grid = (n // grid_r,)

    xs1, xn1 = pl.pallas_call(
        _mm2_kernel,
        grid=grid,
        in_specs=[
            pl.BlockSpec((grid_r, d_in), lambda i: (i, 0)),
            pl.BlockSpec((d_in, d_h), lambda i: (0, 0)),
            pl.BlockSpec((d_in, d_h), lambda i: (0, 0)),
        ],
        out_specs=[
            pl.BlockSpec((grid_r, d_h), lambda i: (i, 0)),
            pl.BlockSpec((grid_r, d_h), lambda i: (i, 0)),
        ],
        out_shape=[
            jax.ShapeDtypeStruct((n, d_h), jnp.float32),
            jax.ShapeDtypeStruct((n, d_h), jnp.float32),
        ],
    )(x, W_self1, W_neigh1)

    degp2 = _make_sc_deg(n_pad, nc2)(idx2, zeros16, ones16)

    agg1p = _make_sc_segsum(n, n_pad, d_half, nc1, True, False)(
        xn1, idx1, zeros_hh, zeros16, ones16)

    hs2, hn2 = pl.pallas_call(
        _layer1_kernel,
        grid=grid,
        in_specs=[
            pl.BlockSpec((grid_r, d_h), lambda i: (i, 0)),
            pl.BlockSpec((None, grid_r, d_half), lambda i: (0, i, 0)),
            pl.BlockSpec((None, grid_r, d_half), lambda i: (1, i, 0)),
            pl.BlockSpec((None, grid_r, DEGW), lambda i: (0, i, 0)),
            pl.BlockSpec((None, grid_r, DEGW), lambda i: (1, i, 0)),
            pl.BlockSpec((1, d_h), lambda i: (0, 0)),
            pl.BlockSpec((d_h, d_out), lambda i: (0, 0)),
            pl.BlockSpec((d_h, d_out), lambda i: (0, 0)),
        ],
        out_specs=[
            pl.BlockSpec((grid_r, d_out), lambda i: (i, 0)),
            pl.BlockSpec((grid_r, d_out), lambda i: (i, 0)),
        ],
        out_shape=[
            jax.ShapeDtypeStruct((n, d_out), jnp.float32),
            jax.ShapeDtypeStruct((n, d_out), jnp.float32),
        ],
    )(xs1, agg1p, agg1p, degp2, degp2, b1.reshape(1, d_h), W_self2, W_neigh2)

    agg2p = _make_sc_segsum(n, n_pad, d_out, nc2, False, False)(
        hn2, idx2, zeros_o, zeros16, ones16)

    out = pl.pallas_call(
        _layer2_kernel,
        grid=grid,
        in_specs=[
            pl.BlockSpec((grid_r, d_out), lambda i: (i, 0)),
            pl.BlockSpec((None, grid_r, d_out), lambda i: (0, i, 0)),
            pl.BlockSpec((None, grid_r, d_out), lambda i: (1, i, 0)),
            pl.BlockSpec((None, grid_r, DEGW), lambda i: (0, i, 0)),
            pl.BlockSpec((None, grid_r, DEGW), lambda i: (1, i, 0)),
            pl.BlockSpec((1, d_out), lambda i: (0, 0)),
        ],
        out_specs=pl.BlockSpec((grid_r, d_out), lambda i: (i, 0)),
        out_shape=jax.ShapeDtypeStruct((n, d_out), jnp.float32),
    )(hs2, agg2p, agg2p, degp2, degp2, b2.reshape(1, d_out))

    return out


# speedup vs baseline: 1.6492x; 1.2644x over previous
import functools

import jax
import jax.numpy as jnp
from jax import lax
from jax.experimental import pallas as pl
from jax.experimental.pallas import tpu as pltpu
from jax.experimental.pallas import tpu_sc as plsc

NCORES = 2
NSUB = 16
NW = NCORES * NSUB
CHUNK = 128
KCH = 40
NBUF = 3
DEGW = 16
SCALE = 256.0


_SC_MESH = plsc.VectorSubcoreMesh(core_axis_name="c", subcore_axis_name="s")
_SC_PARAMS = pltpu.CompilerParams(use_tc_tiling_on_sc=False)


def _make_sc_segsum(n, n_pad, d, n_chunks, col_split, with_deg,
                    dtype=jnp.int16):
    rows_per_sub = n_pad // NSUB
    trows_per_sub = n // NSUB

    out_type = [jax.ShapeDtypeStruct((NCORES, n_pad, d), dtype)]
    scratch = [
        pltpu.VMEM((2 * KCH, CHUNK), jnp.int32),
        [pltpu.VMEM((CHUNK, d), dtype)
         for _ in range(NBUF)],
        pltpu.VMEM_SHARED((n, d), dtype),
        pltpu.VMEM_SHARED((n_pad, d), dtype),
        pltpu.SemaphoreType.DMA,
        pltpu.SemaphoreType.DMA,
    ]
    if with_deg:
        out_type.append(
            jax.ShapeDtypeStruct((NCORES, n_pad, DEGW), jnp.float32))
        scratch += [
            pltpu.VMEM((CHUNK, DEGW), jnp.float32),
            pltpu.VMEM_SHARED((n_pad, DEGW), jnp.float32),
            pltpu.SemaphoreType.DMA,
        ]

    def body(table_hbm, idx_hbm, zeros_hbm, zeros16_hbm, ones_hbm,
             agg_out, *rest):
        if with_deg:
            (deg_out, idx_v, rows, table_sh, agg_sh, gsem, ssem,
             ones_v, deg_sh, dsem) = rest
        else:
            idx_v, rows, table_sh, agg_sh, gsem, ssem = rest
        c = lax.axis_index("c")
        s = lax.axis_index("s")
        sl = pl.ds(s * rows_per_sub, rows_per_sub)
        tsl = pl.ds(s * trows_per_sub, trows_per_sub)
        if col_split:
            pltpu.sync_copy(table_hbm.at[tsl, pl.ds(c * d, d)],
                            table_sh.at[tsl])
        else:
            pltpu.sync_copy(table_hbm.at[tsl], table_sh.at[tsl])
        pltpu.sync_copy(zeros_hbm.at[sl], agg_sh.at[sl])
        if with_deg:
            pltpu.sync_copy(zeros16_hbm.at[sl], deg_sh.at[sl])
            pltpu.sync_copy(ones_hbm, ones_v)
        plsc.subcore_barrier()

        @pl.loop(0, n_chunks // KCH)
        def _(g):
            if col_split:
                pltpu.sync_copy(idx_hbm.at[s, g], idx_v)
            else:
                pltpu.sync_copy(idx_hbm.at[c, s, g], idx_v)
            gathers = [
                pltpu.async_copy(table_sh.at[idx_v.at[0]], rows[0], gsem)]
            scatters = []
            dscats = []
            for j in range(KCH):
                gathers[j].wait()
                scatters.append(pltpu.async_copy(
                    rows[j % NBUF], agg_sh.at[idx_v.at[2 * j + 1]], ssem,
                    add=True))
                if with_deg:
                    dscats.append(pltpu.async_copy(
                        ones_v, deg_sh.at[idx_v.at[2 * j + 1]], dsem,
                        add=True))
                if j + 1 < KCH:
                    if j >= NBUF - 1:
                        scatters[j - (NBUF - 1)].wait()
                    gathers.append(pltpu.async_copy(
                        table_sh.at[idx_v.at[2 * (j + 1)]],
                        rows[(j + 1) % NBUF], gsem))
            for j in range(KCH - NBUF, KCH):
                scatters[j].wait()
            for dsc in dscats:
                dsc.wait()

        plsc.subcore_barrier()
        pltpu.sync_copy(agg_sh.at[sl], agg_out.at[c, sl])
        if with_deg:
            pltpu.sync_copy(deg_sh.at[sl], deg_out.at[c, sl])

    return pl.kernel(body, out_type=tuple(out_type) if with_deg
                     else out_type[0],
                     mesh=_SC_MESH, scratch_types=scratch,
                     compiler_params=_SC_PARAMS)


def _make_sc_deg(n_pad, n_chunks):
    rows_per_sub = n_pad // NSUB

    out_type = jax.ShapeDtypeStruct((NCORES, n_pad, DEGW), jnp.float32)
    scratch = [
        pltpu.VMEM((2 * KCH, CHUNK), jnp.int32),
        pltpu.VMEM((CHUNK, DEGW), jnp.float32),
        pltpu.VMEM_SHARED((n_pad, DEGW), jnp.float32),
        pltpu.SemaphoreType.DMA,
    ]

    def body(idx_hbm, zeros16_hbm, ones_hbm, deg_out,
             idx_v, ones_v, deg_sh, ssem):
        c = lax.axis_index("c")
        s = lax.axis_index("s")
        sl = pl.ds(s * rows_per_sub, rows_per_sub)
        pltpu.sync_copy(zeros16_hbm.at[sl], deg_sh.at[sl])
        pltpu.sync_copy(ones_hbm, ones_v)
        plsc.subcore_barrier()

        @pl.loop(0, n_chunks // KCH)
        def _(g):
            pltpu.sync_copy(idx_hbm.at[c, s, g], idx_v)
            scatters = [
                pltpu.async_copy(ones_v, deg_sh.at[idx_v.at[2 * j + 1]],
                                 ssem, add=True)
                for j in range(KCH)]
            for sc in scatters:
                sc.wait()

        plsc.subcore_barrier()
        pltpu.sync_copy(deg_sh.at[sl], deg_out.at[c, sl])

    return pl.kernel(body, out_type=out_type, mesh=_SC_MESH,
                     scratch_types=scratch, compiler_params=_SC_PARAMS)


def _quant(x):
    return jnp.clip(jnp.round(x * SCALE), -32768.0, 32767.0).astype(jnp.int16)


def _mm2_kernel(x_ref, wa_ref, wb_ref, oa_ref, ob_ref):
    x = x_ref[...]
    oa_ref[...] = jnp.dot(x, wa_ref[...], preferred_element_type=jnp.float32)
    ob_ref[...] = _quant(
        jnp.dot(x, wb_ref[...], preferred_element_type=jnp.float32))


def _layer1_kernel(xs_ref, a0_ref, a1_ref, d0_ref, d1_ref, b_ref,
                   ws2_ref, wn2_ref, hs_ref, hn_ref):
    rdeg = 1.0 / jnp.maximum(d0_ref[:, 0] + d1_ref[:, 0], 1.0)
    aggq = jnp.concatenate([a0_ref[...], a1_ref[...]], axis=1)
    agg = aggq.astype(jnp.float32) * (rdeg * (1.0 / SCALE))[:, None]
    h = jnp.maximum(xs_ref[...] + agg + b_ref[...], 0.0)
    hs_ref[...] = jnp.dot(h, ws2_ref[...], preferred_element_type=jnp.float32)
    hn_ref[...] = _quant(
        jnp.dot(h, wn2_ref[...], preferred_element_type=jnp.float32))


def _layer2_kernel(hs_ref, a0_ref, a1_ref, d0_ref, d1_ref, b_ref, o_ref):
    rdeg = 1.0 / jnp.maximum(d0_ref[:, 0] + d1_ref[:, 0], 1.0)
    aggq = a0_ref[...].astype(jnp.float32) + a1_ref[...].astype(jnp.float32)
    agg = aggq * (rdeg * (1.0 / SCALE))[:, None]
    logits = hs_ref[...] + agg + b_ref[...]
    m = jnp.max(logits, axis=-1, keepdims=True)
    z = logits - m
    lse = jnp.log(jnp.sum(jnp.exp(z), axis=-1, keepdims=True))
    o_ref[...] = z - lse


def kernel(x, edge_index, W_self1, W_neigh1, b1, W_self2, W_neigh2, b2):
    n, d_in = x.shape
    d_h = W_self1.shape[1]
    d_out = W_self2.shape[1]
    e = edge_index.shape[1]

    n_pad = ((n + NSUB - 1) // NSUB + 7) // 8 * 8 * NSUB
    group = KCH * CHUNK
    src_flat = edge_index[0].astype(jnp.int32)
    dst_flat = edge_index[1].astype(jnp.int32)

    def _partition(nw):
        e_per = e // nw
        n_chunks = (e_per + group - 1) // group * KCH
        pad = n_chunks * CHUNK - e_per
        s = jnp.pad(src_flat.reshape(nw, e_per), ((0, 0), (0, pad)))
        d = jnp.pad(dst_flat.reshape(nw, e_per), ((0, 0), (0, pad)),
                    constant_values=n)
        return (s.reshape(nw, n_chunks, CHUNK),
                d.reshape(nw, n_chunks, CHUNK), n_chunks)

    src1, dst1, nc1 = _partition(NSUB)
    src2, dst2, nc2 = _partition(NW)
    idx1 = jnp.stack([src1, dst1], axis=2).reshape(
        NSUB, nc1 // KCH, 2 * KCH, CHUNK)
    idx2 = jnp.stack([src2, dst2], axis=2).reshape(
        NCORES, NSUB, nc2 // KCH, 2 * KCH, CHUNK)

    d_half = d_h // NCORES
    zeros_hh = jnp.zeros((n_pad, d_half), jnp.int16)
    zeros_o = jnp.zeros((n_pad, d_out), jnp.int16)
    zeros16 = jnp.zeros((n_pad, DEGW), jnp.float32)
    ones16 = jnp.ones((CHUNK, DEGW), jnp.float32)

    grid_r = 1000
    grid = (n // grid_r,)

    xs1, xn1 = pl.pallas_call(
        _mm2_kernel,
        grid=grid,
        in_specs=[
            pl.BlockSpec((grid_r, d_in), lambda i: (i, 0)),
            pl.BlockSpec((d_in, d_h), lambda i: (0, 0)),
            pl.BlockSpec((d_in, d_h), lambda i: (0, 0)),
        ],
        out_specs=[
            pl.BlockSpec((grid_r, d_h), lambda i: (i, 0)),
            pl.BlockSpec((grid_r, d_h), lambda i: (i, 0)),
        ],
        out_shape=[
            jax.ShapeDtypeStruct((n, d_h), jnp.float32),
            jax.ShapeDtypeStruct((n, d_h), jnp.int16),
        ],
    )(x, W_self1, W_neigh1)

    degp2 = _make_sc_deg(n_pad, nc2)(idx2, zeros16, ones16)

    agg1p = _make_sc_segsum(n, n_pad, d_half, nc1, True, False)(
        xn1, idx1, zeros_hh, zeros16, ones16)

    hs2, hn2 = pl.pallas_call(
        _layer1_kernel,
        grid=grid,
        in_specs=[
            pl.BlockSpec((grid_r, d_h), lambda i: (i, 0)),
            pl.BlockSpec((None, grid_r, d_half), lambda i: (0, i, 0)),
            pl.BlockSpec((None, grid_r, d_half), lambda i: (1, i, 0)),
            pl.BlockSpec((None, grid_r, DEGW), lambda i: (0, i, 0)),
            pl.BlockSpec((None, grid_r, DEGW), lambda i: (1, i, 0)),
            pl.BlockSpec((1, d_h), lambda i: (0, 0)),
            pl.BlockSpec((d_h, d_out), lambda i: (0, 0)),
            pl.BlockSpec((d_h, d_out), lambda i: (0, 0)),
        ],
        out_specs=[
            pl.BlockSpec((grid_r, d_out), lambda i: (i, 0)),
            pl.BlockSpec((grid_r, d_out), lambda i: (i, 0)),
        ],
        out_shape=[
            jax.ShapeDtypeStruct((n, d_out), jnp.float32),
            jax.ShapeDtypeStruct((n, d_out), jnp.int16),
        ],
    )(xs1, agg1p, agg1p, degp2, degp2, b1.reshape(1, d_h), W_self2, W_neigh2)

    agg2p = _make_sc_segsum(n, n_pad, d_out, nc2, False, False)(
        hn2, idx2, zeros_o, zeros16, ones16)

    out = pl.pallas_call(
        _layer2_kernel,
        grid=grid,
        in_specs=[
            pl.BlockSpec((grid_r, d_out), lambda i: (i, 0)),
            pl.BlockSpec((None, grid_r, d_out), lambda i: (0, i, 0)),
            pl.BlockSpec((None, grid_r, d_out), lambda i: (1, i, 0)),
            pl.BlockSpec((None, grid_r, DEGW), lambda i: (0, i, 0)),
            pl.BlockSpec((None, grid_r, DEGW), lambda i: (1, i, 0)),
            pl.BlockSpec((1, d_out), lambda i: (0, 0)),
        ],
        out_specs=pl.BlockSpec((grid_r, d_out), lambda i: (i, 0)),
        out_shape=jax.ShapeDtypeStruct((n, d_out), jnp.float32),
    )(hs2, agg2p, agg2p, degp2, degp2, b2.reshape(1, d_out))

    return out


# speedup vs baseline: 1.8615x; 1.1288x over previous
import functools

import jax
import jax.numpy as jnp
from jax import lax
from jax.experimental import pallas as pl
from jax.experimental.pallas import tpu as pltpu
from jax.experimental.pallas import tpu_sc as plsc

NCORES = 2
NSUB = 16
NW = NCORES * NSUB
CHUNK = 128
KCH = 39
NBUF = 3
DEGW = 16
SCALE = 256.0


_SC_MESH = plsc.VectorSubcoreMesh(core_axis_name="c", subcore_axis_name="s")
_SC_PARAMS = pltpu.CompilerParams(use_tc_tiling_on_sc=False)


def _zero_fill(zrow_v, dst_sh, base, nrows):
    off = 0
    while off + CHUNK <= nrows:
        pltpu.sync_copy(zrow_v, dst_sh.at[pl.ds(base + off, CHUNK)])
        off += CHUNK
    if off < nrows:
        pltpu.sync_copy(zrow_v.at[pl.ds(0, nrows - off)],
                        dst_sh.at[pl.ds(base + off, nrows - off)])


def _make_sc_segsum(n, n_pad, d, n_full, n_tail, col_split,
                    dtype=jnp.int16):
    rows_per_sub = n_pad // NSUB
    trows_per_sub = n // NSUB
    lanes = 32 if dtype == jnp.int16 else 16

    out_type = jax.ShapeDtypeStruct((NCORES, n_pad, d), dtype)
    scratch = [
        pltpu.VMEM((KCH, CHUNK), jnp.int32),
        pltpu.VMEM((KCH, CHUNK), jnp.int32),
        pltpu.VMEM((n_tail,), jnp.int32),
        pltpu.VMEM((n_tail,), jnp.int32),
        [pltpu.VMEM((CHUNK, d), dtype)
         for _ in range(NBUF)],
        pltpu.VMEM((CHUNK, d), dtype),
        pltpu.VMEM_SHARED((n, d), dtype),
        pltpu.VMEM_SHARED((n_pad, d), dtype),
        pltpu.SemaphoreType.DMA,
        pltpu.SemaphoreType.DMA,
    ]

    def body(table_hbm, idx_hbm, tail_hbm, agg_out,
             src_v, dst_v, tsrc_v, tdst_v, rows, zrow_v, table_sh, agg_sh,
             gsem, ssem):
        c = lax.axis_index("c")
        s = lax.axis_index("s")
        w = s if col_split else c * NSUB + s
        sl = pl.ds(s * rows_per_sub, rows_per_sub)
        tsl = pl.ds(s * trows_per_sub, trows_per_sub)
        if col_split:
            tstage = pltpu.async_copy(table_hbm.at[tsl, pl.ds(c * d, d)],
                                      table_sh.at[tsl], gsem)
        else:
            tstage = pltpu.async_copy(table_hbm.at[tsl], table_sh.at[tsl],
                                      gsem)
        zero = jnp.zeros((lanes,), dtype)

        @pl.loop(0, CHUNK)
        def _(r):
            for k in range(d // lanes):
                zrow_v[r, pl.ds(k * lanes, lanes)] = zero

        _zero_fill(zrow_v, agg_sh, s * rows_per_sub, rows_per_sub)
        tstage.wait()
        plsc.subcore_barrier()

        @pl.loop(0, n_full // KCH)
        def _(g):
            gsl = pl.ds(g * KCH, KCH)
            pltpu.sync_copy(idx_hbm.at[0, w, gsl], src_v)
            pltpu.sync_copy(idx_hbm.at[1, w, gsl], dst_v)
            gathers = [
                pltpu.async_copy(table_sh.at[src_v.at[0]], rows[0], gsem)]
            scatters = []
            for j in range(KCH):
                gathers[j].wait()
                scatters.append(pltpu.async_copy(
                    rows[j % NBUF], agg_sh.at[dst_v.at[j]], ssem, add=True))
                if j + 1 < KCH:
                    if j >= NBUF - 1:
                        scatters[j - (NBUF - 1)].wait()
                    gathers.append(pltpu.async_copy(
                        table_sh.at[src_v.at[j + 1]],
                        rows[(j + 1) % NBUF], gsem))
            for j in range(KCH - NBUF, KCH):
                scatters[j].wait()

        pltpu.sync_copy(tail_hbm.at[0, w], tsrc_v)
        pltpu.sync_copy(tail_hbm.at[1, w], tdst_v)
        tail_rows = rows[0].at[pl.ds(0, n_tail)]
        pltpu.async_copy(table_sh.at[tsrc_v], tail_rows, gsem).wait()
        pltpu.async_copy(tail_rows, agg_sh.at[tdst_v], ssem, add=True).wait()

        plsc.subcore_barrier()
        pltpu.sync_copy(agg_sh.at[sl], agg_out.at[c, sl])

    return pl.kernel(body, out_type=out_type, mesh=_SC_MESH,
                     scratch_types=scratch, compiler_params=_SC_PARAMS)


def _make_sc_deg(n_pad, n_full, n_tail):
    rows_per_sub = n_pad // NSUB

    out_type = jax.ShapeDtypeStruct((NCORES, n_pad, DEGW), jnp.float32)
    scratch = [
        pltpu.VMEM((KCH, CHUNK), jnp.int32),
        pltpu.VMEM((n_tail,), jnp.int32),
        pltpu.VMEM((CHUNK, DEGW), jnp.float32),
        pltpu.VMEM((CHUNK, DEGW), jnp.float32),
        pltpu.VMEM_SHARED((n_pad, DEGW), jnp.float32),
        pltpu.SemaphoreType.DMA,
    ]

    def body(idx_hbm, tail_hbm, deg_out,
             dst_v, tdst_v, ones_v, zrow_v, deg_sh, ssem):
        c = lax.axis_index("c")
        s = lax.axis_index("s")
        w = c * NSUB + s
        sl = pl.ds(s * rows_per_sub, rows_per_sub)
        one = jnp.ones((DEGW,), jnp.float32)
        zero = jnp.zeros((DEGW,), jnp.float32)

        @pl.loop(0, CHUNK)
        def _(r):
            ones_v[r, :] = one
            zrow_v[r, :] = zero

        _zero_fill(zrow_v, deg_sh, s * rows_per_sub, rows_per_sub)
        plsc.subcore_barrier()

        @pl.loop(0, n_full // KCH)
        def _(g):
            pltpu.sync_copy(idx_hbm.at[1, w, pl.ds(g * KCH, KCH)], dst_v)
            scatters = [
                pltpu.async_copy(ones_v, deg_sh.at[dst_v.at[j]],
                                 ssem, add=True)
                for j in range(KCH)]
            for sc in scatters:
                sc.wait()

        pltpu.sync_copy(tail_hbm.at[1, w], tdst_v)
        pltpu.async_copy(ones_v.at[pl.ds(0, n_tail)], deg_sh.at[tdst_v],
                         ssem, add=True).wait()

        plsc.subcore_barrier()
        pltpu.sync_copy(deg_sh.at[sl], deg_out.at[c, sl])

    return pl.kernel(body, out_type=out_type, mesh=_SC_MESH,
                     scratch_types=scratch, compiler_params=_SC_PARAMS)


def _quant(x):
    return jnp.clip(jnp.round(x * SCALE), -32768.0, 32767.0).astype(jnp.int16)


def _mm2_kernel(x_ref, wa_ref, wb_ref, oa_ref, ob_ref):
    x = x_ref[...]
    oa_ref[...] = jnp.dot(x, wa_ref[...], preferred_element_type=jnp.float32)
    ob_ref[...] = _quant(
        jnp.dot(x, wb_ref[...], preferred_element_type=jnp.float32))


def _layer1_kernel(xs_ref, a0_ref, a1_ref, d0_ref, d1_ref, b_ref,
                   ws2_ref, wn2_ref, hs_ref, hn_ref):
    rdeg = 1.0 / jnp.maximum(d0_ref[:, 0] + d1_ref[:, 0], 1.0)
    aggq = jnp.concatenate([a0_ref[...], a1_ref[...]], axis=1)
    agg = aggq.astype(jnp.float32) * (rdeg * (1.0 / SCALE))[:, None]
    h = jnp.maximum(xs_ref[...] + agg + b_ref[...], 0.0)
    hs_ref[...] = jnp.dot(h, ws2_ref[...], preferred_element_type=jnp.float32)
    hn_ref[...] = _quant(
        jnp.dot(h, wn2_ref[...], preferred_element_type=jnp.float32))


def _layer2_kernel(hs_ref, a0_ref, a1_ref, d0_ref, d1_ref, b_ref, o_ref):
    rdeg = 1.0 / jnp.maximum(d0_ref[:, 0] + d1_ref[:, 0], 1.0)
    aggq = a0_ref[...].astype(jnp.float32) + a1_ref[...].astype(jnp.float32)
    agg = aggq * (rdeg * (1.0 / SCALE))[:, None]
    logits = hs_ref[...] + agg + b_ref[...]
    m = jnp.max(logits, axis=-1, keepdims=True)
    z = logits - m
    lse = jnp.log(jnp.sum(jnp.exp(z), axis=-1, keepdims=True))
    o_ref[...] = z - lse


def kernel(x, edge_index, W_self1, W_neigh1, b1, W_self2, W_neigh2, b2):
    n, d_in = x.shape
    d_h = W_self1.shape[1]
    d_out = W_self2.shape[1]
    e = edge_index.shape[1]

    n_pad = ((n + NSUB - 1) // NSUB + 7) // 8 * 8 * NSUB
    ei = edge_index.astype(jnp.int32)

    def _partition(nw):
        e_per = e // nw
        n_full = e_per // CHUNK // KCH * KCH
        n_tail = e_per - n_full * CHUNK
        r = ei.reshape(2, nw, e_per)
        idx = r[:, :, :n_full * CHUNK].reshape(2, nw, n_full, CHUNK)
        tail = r[:, :, n_full * CHUNK:]
        return idx, tail, n_full, n_tail

    idx1, tail1, nf1, nt1 = _partition(NSUB)
    idx2, tail2, nf2, nt2 = _partition(NW)

    d_half = d_h // NCORES

    grid_r = 1000
    grid = (n // grid_r,)

    xs1, xn1 = pl.pallas_call(
        _mm2_kernel,
        grid=grid,
        in_specs=[
            pl.BlockSpec((grid_r, d_in), lambda i: (i, 0)),
            pl.BlockSpec((d_in, d_h), lambda i: (0, 0)),
            pl.BlockSpec((d_in, d_h), lambda i: (0, 0)),
        ],
        out_specs=[
            pl.BlockSpec((grid_r, d_h), lambda i: (i, 0)),
            pl.BlockSpec((grid_r, d_h), lambda i: (i, 0)),
        ],
        out_shape=[
            jax.ShapeDtypeStruct((n, d_h), jnp.float32),
            jax.ShapeDtypeStruct((n, d_h), jnp.int16),
        ],
    )(x, W_self1, W_neigh1)

    degp2 = _make_sc_deg(n_pad, nf2, nt2)(idx2, tail2)

    agg1p = _make_sc_segsum(n, n_pad, d_half, nf1, nt1, True)(
        xn1, idx1, tail1)

    hs2, hn2 = pl.pallas_call(
        _layer1_kernel,
        grid=grid,
        in_specs=[
            pl.BlockSpec((grid_r, d_h), lambda i: (i, 0)),
            pl.BlockSpec((None, grid_r, d_half), lambda i: (0, i, 0)),
            pl.BlockSpec((None, grid_r, d_half), lambda i: (1, i, 0)),
            pl.BlockSpec((None, grid_r, DEGW), lambda i: (0, i, 0)),
            pl.BlockSpec((None, grid_r, DEGW), lambda i: (1, i, 0)),
            pl.BlockSpec((1, d_h), lambda i: (0, 0)),
            pl.BlockSpec((d_h, d_out), lambda i: (0, 0)),
            pl.BlockSpec((d_h, d_out), lambda i: (0, 0)),
        ],
        out_specs=[
            pl.BlockSpec((grid_r, d_out), lambda i: (i, 0)),
            pl.BlockSpec((grid_r, d_out), lambda i: (i, 0)),
        ],
        out_shape=[
            jax.ShapeDtypeStruct((n, d_out), jnp.float32),
            jax.ShapeDtypeStruct((n, d_out), jnp.int16),
        ],
    )(xs1, agg1p, agg1p, degp2, degp2, b1.reshape(1, d_h), W_self2, W_neigh2)

    agg2p = _make_sc_segsum(n, n_pad, d_out, nf2, nt2, False)(
        hn2, idx2, tail2)

    out = pl.pallas_call(
        _layer2_kernel,
        grid=grid,
        in_specs=[
            pl.BlockSpec((grid_r, d_out), lambda i: (i, 0)),
            pl.BlockSpec((None, grid_r, d_out), lambda i: (0, i, 0)),
            pl.BlockSpec((None, grid_r, d_out), lambda i: (1, i, 0)),
            pl.BlockSpec((None, grid_r, DEGW), lambda i: (0, i, 0)),
            pl.BlockSpec((None, grid_r, DEGW), lambda i: (1, i, 0)),
            pl.BlockSpec((1, d_out), lambda i: (0, 0)),
        ],
        out_specs=pl.BlockSpec((grid_r, d_out), lambda i: (i, 0)),
        out_shape=jax.ShapeDtypeStruct((n, d_out), jnp.float32),
    )(hs2, agg2p, agg2p, degp2, degp2, b2.reshape(1, d_out))

    return out
